# baseline (speedup 1.0000x reference)
import functools

import jax
import jax.numpy as jnp
from jax import lax
from jax.experimental import pallas as pl
from jax.experimental.pallas import tpu as pltpu
from jax.experimental.pallas import tpu_sc as plsc

BLK = 256


def _dist_argmin_body(z_ref, zsq_ref, esq_ref, et_ref, idx_ref):
    z = z_ref[...]
    e_t = et_ref[...]
    k_dim = e_t.shape[1]

    dot = lax.dot_general(
        z, e_t, dimension_numbers=(((1,), (0,)), ((), ())),
        precision=lax.Precision.DEFAULT)
    d2 = (zsq_ref[...] + esq_ref[...]) - 2.0 * dot
    dist = jnp.sqrt(jnp.maximum(d2, 0.0))

    minval = jnp.min(dist, axis=1, keepdims=True)
    iota_k = lax.broadcasted_iota(jnp.int32, dist.shape, 1)
    cand = jnp.where(dist == minval, iota_k, k_dim)
    idx_ref[0, 0, :] = jnp.min(cand, axis=1)


def _st_loss_body(z_ref, zq_ref, zqst_ref, loss_ref):
    i = pl.program_id(0)
    z = z_ref[...]
    z_q = zq_ref[...]
    diff = z_q - z
    zqst_ref[...] = z + diff
    psum = jnp.sum(diff * diff)

    @pl.when(i == 0)
    def _():
        loss_ref[...] = jnp.zeros_like(loss_ref)

    loss_ref[...] += psum


def _make_sc_gather(b_dim, d_pad, n_workers):
    b_per_w = b_dim // n_workers
    mesh = plsc.VectorSubcoreMesh(core_axis_name="c", subcore_axis_name="s")

    @functools.partial(
        pl.kernel, mesh=mesh,
        out_type=jax.ShapeDtypeStruct((b_dim, d_pad), jnp.float32),
        scratch_types=[
            pltpu.VMEM((b_per_w,), jnp.int32),
            pltpu.VMEM((b_per_w, d_pad), jnp.float32),
            pltpu.SemaphoreType.DMA,
        ],
    )
    def gather(emb_hbm, idx_hbm, out_hbm, idx_v, rows_v, sem):
        wid = lax.axis_index("s") * 2 + lax.axis_index("c")
        base = wid * b_per_w
        pltpu.sync_copy(idx_hbm.at[pl.ds(base, b_per_w)], idx_v)
        pltpu.async_copy(emb_hbm.at[idx_v], rows_v, sem).wait()
        pltpu.sync_copy(rows_v, out_hbm.at[pl.ds(base, b_per_w)])

    return gather


def kernel(z, embeddings):
    b_dim, d_dim = z.shape
    k_dim = embeddings.shape[0]
    nblk = b_dim // BLK

    z_sq = jnp.sum(z * z, axis=1, keepdims=True)
    e_sq = jnp.sum(embeddings * embeddings, axis=1)[None, :]
    e_t = embeddings.T

    idx3 = pl.pallas_call(
        _dist_argmin_body,
        grid=(nblk,),
        in_specs=[
            pl.BlockSpec((BLK, d_dim), lambda i: (i, 0)),
            pl.BlockSpec((BLK, 1), lambda i: (i, 0)),
            pl.BlockSpec((1, k_dim), lambda i: (0, 0)),
            pl.BlockSpec((d_dim, k_dim), lambda i: (0, 0)),
        ],
        out_specs=pl.BlockSpec((1, 1, BLK), lambda i: (i, 0, 0)),
        out_shape=jax.ShapeDtypeStruct((nblk, 1, BLK), jnp.int32),
    )(z, z_sq, e_sq, e_t)
    indices = idx3.reshape(b_dim)

    emb_pad = jnp.pad(embeddings, ((0, 0), (0, 128 - d_dim)))
    z_q = _make_sc_gather(b_dim, 128, 32)(emb_pad, indices)[:, :d_dim]

    zqst, loss_acc = pl.pallas_call(
        _st_loss_body,
        grid=(nblk,),
        in_specs=[
            pl.BlockSpec((BLK, d_dim), lambda i: (i, 0)),
            pl.BlockSpec((BLK, d_dim), lambda i: (i, 0)),
        ],
        out_specs=[
            pl.BlockSpec((BLK, d_dim), lambda i: (i, 0)),
            pl.BlockSpec((8, 128), lambda i: (0, 0)),
        ],
        out_shape=[
            jax.ShapeDtypeStruct((b_dim, d_dim), jnp.float32),
            jax.ShapeDtypeStruct((8, 128), jnp.float32),
        ],
    )(z, z_q)

    vq_loss = 1.25 * loss_acc[0, 0] / (b_dim * d_dim)
    return zqst, indices, vq_loss


# speedup vs baseline: 1.5681x; 1.5681x over previous
import functools

import jax
import jax.numpy as jnp
from jax import lax
from jax.experimental import pallas as pl
from jax.experimental.pallas import tpu as pltpu
from jax.experimental.pallas import tpu_sc as plsc

BLK = 256


def _dist_argmin_body(z_ref, esq_ref, et2_ref, idx_ref):
    z = z_ref[...]
    et2 = et2_ref[...]
    k_dim = et2.shape[1]

    s = lax.dot_general(
        z, et2, dimension_numbers=(((1,), (0,)), ((), ())),
        precision=lax.Precision.DEFAULT) + esq_ref[...]

    minval = jnp.min(s, axis=1, keepdims=True)
    iota_k = lax.broadcasted_iota(jnp.int32, s.shape, 1)
    cand = jnp.where(s == minval, iota_k, k_dim)
    idx_ref[0, 0, :] = jnp.min(cand, axis=1)


def _st_loss_body(z_ref, zq_ref, zqst_ref, loss_ref):
    i = pl.program_id(0)
    z = z_ref[...]
    z_q = zq_ref[...]
    diff = z_q - z
    zqst_ref[...] = z + diff
    psum = jnp.sum(diff * diff)

    @pl.when(i == 0)
    def _():
        loss_ref[...] = jnp.zeros_like(loss_ref)

    loss_ref[...] += psum


def _make_sc_gather(b_dim, d_pad, n_workers):
    b_per_w = b_dim // n_workers
    mesh = plsc.VectorSubcoreMesh(core_axis_name="c", subcore_axis_name="s")

    @functools.partial(
        pl.kernel, mesh=mesh,
        out_type=jax.ShapeDtypeStruct((b_dim, d_pad), jnp.float32),
        scratch_types=[
            pltpu.VMEM((b_per_w,), jnp.int32),
            pltpu.VMEM((b_per_w, d_pad), jnp.float32),
            pltpu.SemaphoreType.DMA,
        ],
    )
    def gather(emb_hbm, idx_hbm, out_hbm, idx_v, rows_v, sem):
        wid = lax.axis_index("s") * 2 + lax.axis_index("c")
        base = wid * b_per_w
        pltpu.sync_copy(idx_hbm.at[pl.ds(base, b_per_w)], idx_v)
        pltpu.async_copy(emb_hbm.at[idx_v], rows_v, sem).wait()
        pltpu.sync_copy(rows_v, out_hbm.at[pl.ds(base, b_per_w)])

    return gather


def kernel(z, embeddings):
    b_dim, d_dim = z.shape
    k_dim = embeddings.shape[0]
    nblk = b_dim // BLK

    e_sq = jnp.sum(embeddings * embeddings, axis=1)[None, :]
    et2 = embeddings.T * -2.0

    idx3 = pl.pallas_call(
        _dist_argmin_body,
        grid=(nblk,),
        in_specs=[
            pl.BlockSpec((BLK, d_dim), lambda i: (i, 0)),
            pl.BlockSpec((1, k_dim), lambda i: (0, 0)),
            pl.BlockSpec((d_dim, k_dim), lambda i: (0, 0)),
        ],
        out_specs=pl.BlockSpec((1, 1, BLK), lambda i: (i, 0, 0)),
        out_shape=jax.ShapeDtypeStruct((nblk, 1, BLK), jnp.int32),
    )(z, e_sq, et2)
    indices = idx3.reshape(b_dim)

    emb_pad = jnp.pad(embeddings, ((0, 0), (0, 128 - d_dim)))
    z_q = _make_sc_gather(b_dim, 128, 32)(emb_pad, indices)[:, :d_dim]

    zqst, loss_acc = pl.pallas_call(
        _st_loss_body,
        grid=(nblk,),
        in_specs=[
            pl.BlockSpec((BLK, d_dim), lambda i: (i, 0)),
            pl.BlockSpec((BLK, d_dim), lambda i: (i, 0)),
        ],
        out_specs=[
            pl.BlockSpec((BLK, d_dim), lambda i: (i, 0)),
            pl.BlockSpec((8, 128), lambda i: (0, 0)),
        ],
        out_shape=[
            jax.ShapeDtypeStruct((b_dim, d_dim), jnp.float32),
            jax.ShapeDtypeStruct((8, 128), jnp.float32),
        ],
    )(z, z_q)

    vq_loss = 1.25 * loss_acc[0, 0] / (b_dim * d_dim)
    return zqst, indices, vq_loss


# speedup vs baseline: 1.6661x; 1.0625x over previous
import functools

import jax
import jax.numpy as jnp
from jax import lax
from jax.experimental import pallas as pl
from jax.experimental.pallas import tpu as pltpu
from jax.experimental.pallas import tpu_sc as plsc

BLK = 512


def _dist_argmin_body(z_ref, esq_ref, et2_ref, idx_ref):
    z = z_ref[...]
    et2 = et2_ref[...]
    k_dim = et2.shape[1]

    s = lax.dot_general(
        z, et2, dimension_numbers=(((1,), (0,)), ((), ())),
        precision=lax.Precision.DEFAULT) + esq_ref[...]

    minval = jnp.min(s, axis=1, keepdims=True)
    iota_k = lax.broadcasted_iota(jnp.int32, s.shape, 1)
    cand = jnp.where(s == minval, iota_k, k_dim)
    idx_ref[0, 0, :] = jnp.min(cand, axis=1)


def _st_loss_body(z_ref, zq_ref, zqst_ref, loss_ref):
    i = pl.program_id(0)
    z = z_ref[...]
    z_q = zq_ref[...]
    diff = z_q - z
    zqst_ref[...] = z + diff
    psum = jnp.sum(diff * diff)

    @pl.when(i == 0)
    def _():
        loss_ref[...] = jnp.zeros_like(loss_ref)

    loss_ref[...] += psum


def _make_sc_gather(b_dim, d_pad, n_workers):
    b_per_w = b_dim // n_workers
    mesh = plsc.VectorSubcoreMesh(core_axis_name="c", subcore_axis_name="s")

    @functools.partial(
        pl.kernel, mesh=mesh,
        out_type=jax.ShapeDtypeStruct((b_dim, d_pad), jnp.float32),
        scratch_types=[
            pltpu.VMEM((b_per_w,), jnp.int32),
            pltpu.VMEM((b_per_w, d_pad), jnp.float32),
            pltpu.SemaphoreType.DMA,
        ],
    )
    def gather(emb_hbm, idx_hbm, out_hbm, idx_v, rows_v, sem):
        wid = lax.axis_index("s") * 2 + lax.axis_index("c")
        base = wid * b_per_w
        pltpu.sync_copy(idx_hbm.at[pl.ds(base, b_per_w)], idx_v)
        pltpu.async_copy(emb_hbm.at[idx_v], rows_v, sem).wait()
        pltpu.sync_copy(rows_v, out_hbm.at[pl.ds(base, b_per_w)])

    return gather


def kernel(z, embeddings):
    b_dim, d_dim = z.shape
    k_dim = embeddings.shape[0]
    nblk = b_dim // BLK

    e_sq = jnp.sum(embeddings * embeddings, axis=1)[None, :]
    et2 = embeddings.T * -2.0

    idx3 = pl.pallas_call(
        _dist_argmin_body,
        grid=(nblk,),
        in_specs=[
            pl.BlockSpec((BLK, d_dim), lambda i: (i, 0)),
            pl.BlockSpec((1, k_dim), lambda i: (0, 0)),
            pl.BlockSpec((d_dim, k_dim), lambda i: (0, 0)),
        ],
        out_specs=pl.BlockSpec((1, 1, BLK), lambda i: (i, 0, 0)),
        out_shape=jax.ShapeDtypeStruct((nblk, 1, BLK), jnp.int32),
    )(z, e_sq, et2)
    indices = idx3.reshape(b_dim)

    emb_pad = jnp.pad(embeddings, ((0, 0), (0, 128 - d_dim)))
    z_q = _make_sc_gather(b_dim, 128, 32)(emb_pad, indices)[:, :d_dim]

    zqst, loss_acc = pl.pallas_call(
        _st_loss_body,
        grid=(nblk,),
        in_specs=[
            pl.BlockSpec((BLK, d_dim), lambda i: (i, 0)),
            pl.BlockSpec((BLK, d_dim), lambda i: (i, 0)),
        ],
        out_specs=[
            pl.BlockSpec((BLK, d_dim), lambda i: (i, 0)),
            pl.BlockSpec((8, 128), lambda i: (0, 0)),
        ],
        out_shape=[
            jax.ShapeDtypeStruct((b_dim, d_dim), jnp.float32),
            jax.ShapeDtypeStruct((8, 128), jnp.float32),
        ],
    )(z, z_q)

    vq_loss = 1.25 * loss_acc[0, 0] / (b_dim * d_dim)
    return zqst, indices, vq_loss
